# baseline (speedup 1.0000x reference)
import functools

import jax
import jax.numpy as jnp
from jax.experimental import pallas as pl
from jax.experimental.pallas import tpu as pltpu


def _gate_kernel(x_ref, w_ref, idx_ref, score_ref, loss_ref, acc_ref,
                 *, nsteps, n_tokens, n_experts):
    i = pl.program_id(0)

    xb = x_ref[...]
    w = w_ref[...]
    logits = jax.lax.dot_general(
        xb, w, (((1,), (1,)), ((), ())),
        preferred_element_type=jnp.float32)

    m = jnp.max(logits, axis=-1, keepdims=True)
    p = jnp.exp(logits - m)
    s = jnp.sum(p, axis=-1, keepdims=True)
    probs = p / s

    score = jnp.max(probs, axis=-1)
    t, e = probs.shape
    iota = jax.lax.broadcasted_iota(jnp.int32, (t, e), 1)
    is_max = probs == score[:, None]
    idx = jnp.min(jnp.where(is_max, iota, e), axis=-1)

    idx_ref[...] = idx
    score_ref[...] = score

    onehot = (iota == idx[:, None]).astype(jnp.float32)

    @pl.when(i == 0)
    def _init():
        acc_ref[...] = jnp.zeros_like(acc_ref)

    acc_ref[0, :] += jnp.sum(probs, axis=0)
    acc_ref[1, :] += jnp.sum(onehot, axis=0)

    @pl.when(i == nsteps - 1)
    def _finish():
        psum = acc_ref[0, :]
        cnt = acc_ref[1, :]
        loss = (n_experts / (n_tokens * n_tokens)) * jnp.sum(psum * cnt)
        loss_ref[...] = jnp.full((1, 1), loss, dtype=jnp.float32)


def kernel(x, W):
    b, s, d = x.shape
    e = W.shape[0]
    n = b * s
    x2 = x.reshape(n, d)

    t = 1024
    nsteps = n // t

    body = functools.partial(_gate_kernel, nsteps=nsteps,
                             n_tokens=n, n_experts=e)

    idx, score, loss = pl.pallas_call(
        body,
        grid=(nsteps,),
        in_specs=[
            pl.BlockSpec((t, d), lambda i: (i, 0)),
            pl.BlockSpec((e, d), lambda i: (0, 0)),
        ],
        out_specs=[
            pl.BlockSpec((t,), lambda i: (i,)),
            pl.BlockSpec((t,), lambda i: (i,)),
            pl.BlockSpec((1, 1), lambda i: (0, 0)),
        ],
        out_shape=[
            jax.ShapeDtypeStruct((n,), jnp.int32),
            jax.ShapeDtypeStruct((n,), jnp.float32),
            jax.ShapeDtypeStruct((1, 1), jnp.float32),
        ],
        scratch_shapes=[pltpu.VMEM((2, e), jnp.float32)],
    )(x2, W)

    return idx, score, loss.reshape(())


# speedup vs baseline: 1.6796x; 1.6796x over previous
import functools

import jax
import jax.numpy as jnp
from jax.experimental import pallas as pl
from jax.experimental.pallas import tpu as pltpu


def _gate_kernel(x_ref, w_ref, idx_ref, score_ref, loss_ref,
                 logits_ref, psum_ref, cnt_ref,
                 *, nsteps, n_tokens, n_experts, t):
    i = pl.program_id(0)
    e = n_experts

    @pl.when(i == 0)
    def _init():
        psum_ref[...] = jnp.zeros_like(psum_ref)
        cnt_ref[...] = jnp.zeros_like(cnt_ref)

    @pl.when(i < nsteps)
    def _matmul():
        xb = x_ref[...]
        w = w_ref[...]
        logits_ref[i % 2] = jax.lax.dot_general(
            w, xb, (((1,), (1,)), ((), ())),
            preferred_element_type=jnp.float32)

    @pl.when(i > 0)
    def _epilogue():
        logits = logits_ref[(i - 1) % 2]
        m = jnp.max(logits, axis=0, keepdims=True)
        p = jnp.exp(logits - m)
        s = jnp.sum(p, axis=0, keepdims=True)
        recip = 1.0 / s
        probs = p * recip

        onehot = probs == recip
        iota = jax.lax.broadcasted_iota(jnp.int32, (e, t), 0)
        idx = jnp.min(jnp.where(onehot, iota, e), axis=0)

        idx_ref[...] = idx
        score_ref[...] = recip[0]
        psum_ref[...] += probs
        cnt_ref[...] += onehot.astype(jnp.float32)

    @pl.when(i == nsteps)
    def _finish():
        psum = jnp.sum(psum_ref[...], axis=1)
        cnt = jnp.sum(cnt_ref[...], axis=1)
        loss = (e / (n_tokens * n_tokens)) * jnp.sum(psum * cnt)
        loss_ref[...] = jnp.full((1, 1), loss, dtype=jnp.float32)


def kernel(x, W):
    b, s, d = x.shape
    e = W.shape[0]
    n = b * s
    x2 = x.reshape(n, d)

    t = 1024
    nsteps = n // t

    body = functools.partial(_gate_kernel, nsteps=nsteps,
                             n_tokens=n, n_experts=e, t=t)

    idx, score, loss = pl.pallas_call(
        body,
        grid=(nsteps + 1,),
        in_specs=[
            pl.BlockSpec((t, d), lambda i: (jnp.minimum(i, nsteps - 1), 0)),
            pl.BlockSpec((e, d), lambda i: (0, 0)),
        ],
        out_specs=[
            pl.BlockSpec((t,), lambda i: (jnp.maximum(i - 1, 0),)),
            pl.BlockSpec((t,), lambda i: (jnp.maximum(i - 1, 0),)),
            pl.BlockSpec((1, 1), lambda i: (0, 0)),
        ],
        out_shape=[
            jax.ShapeDtypeStruct((n,), jnp.int32),
            jax.ShapeDtypeStruct((n,), jnp.float32),
            jax.ShapeDtypeStruct((1, 1), jnp.float32),
        ],
        scratch_shapes=[
            pltpu.VMEM((2, e, t), jnp.float32),
            pltpu.VMEM((e, t), jnp.float32),
            pltpu.VMEM((e, t), jnp.float32),
        ],
    )(x2, W)

    return idx, score, loss.reshape(())


# speedup vs baseline: 1.6831x; 1.0021x over previous
import functools

import jax
import jax.numpy as jnp
from jax.experimental import pallas as pl
from jax.experimental.pallas import tpu as pltpu


def _gate_kernel(x_ref, w_ref, idx_ref, score_ref, loss_ref,
                 logits_ref, psum_ref, cnt_ref,
                 *, nsteps, n_tokens, n_experts, t):
    i = pl.program_id(0)
    e = n_experts

    @pl.when(i == 0)
    def _init():
        psum_ref[...] = jnp.zeros_like(psum_ref)
        cnt_ref[...] = jnp.zeros_like(cnt_ref)

    @pl.when(i < nsteps)
    def _matmul():
        xb = x_ref[...]
        w = w_ref[...]
        logits_ref[i % 2] = jax.lax.dot_general(
            w, xb, (((1,), (1,)), ((), ())),
            preferred_element_type=jnp.float32)

    @pl.when(i > 0)
    def _epilogue():
        logits = logits_ref[(i - 1) % 2]
        m = jnp.max(logits, axis=0, keepdims=True)
        p = jnp.exp(logits - m)
        s = jnp.sum(p, axis=0, keepdims=True)
        recip = 1.0 / s
        probs = p * recip

        onehot = probs == recip
        iota = jax.lax.broadcasted_iota(jnp.int32, (e, t), 0)
        idx = jnp.min(jnp.where(onehot, iota, e), axis=0)

        idx_ref[...] = idx
        score_ref[...] = recip[0]
        psum_ref[...] += probs
        cnt_ref[...] += onehot.astype(jnp.float32)

    @pl.when(i == nsteps)
    def _finish():
        psum = jnp.sum(psum_ref[...], axis=1)
        cnt = jnp.sum(cnt_ref[...], axis=1)
        loss = (e / (n_tokens * n_tokens)) * jnp.sum(psum * cnt)
        loss_ref[...] = jnp.full((1, 1), loss, dtype=jnp.float32)


def kernel(x, W):
    b, s, d = x.shape
    e = W.shape[0]
    n = b * s
    x2 = x.reshape(n, d)

    t = 2048
    nsteps = n // t

    body = functools.partial(_gate_kernel, nsteps=nsteps,
                             n_tokens=n, n_experts=e, t=t)

    idx, score, loss = pl.pallas_call(
        body,
        grid=(nsteps + 1,),
        in_specs=[
            pl.BlockSpec((t, d), lambda i: (jnp.minimum(i, nsteps - 1), 0)),
            pl.BlockSpec((e, d), lambda i: (0, 0)),
        ],
        out_specs=[
            pl.BlockSpec((t,), lambda i: (jnp.maximum(i - 1, 0),)),
            pl.BlockSpec((t,), lambda i: (jnp.maximum(i - 1, 0),)),
            pl.BlockSpec((1, 1), lambda i: (0, 0)),
        ],
        out_shape=[
            jax.ShapeDtypeStruct((n,), jnp.int32),
            jax.ShapeDtypeStruct((n,), jnp.float32),
            jax.ShapeDtypeStruct((1, 1), jnp.float32),
        ],
        scratch_shapes=[
            pltpu.VMEM((2, e, t), jnp.float32),
            pltpu.VMEM((e, t), jnp.float32),
            pltpu.VMEM((e, t), jnp.float32),
        ],
    )(x2, W)

    return idx, score, loss.reshape(())
